# baseline (speedup 1.0000x reference)
import functools

import jax
import jax.numpy as jnp
from jax import lax
from jax.experimental import pallas as pl
from jax.experimental.pallas import tpu as pltpu
from jax.experimental.pallas import tpu_sc as plsc

N = 10000
E = 320000
NP = 10240
NC = 2
NS = 16
NW = NC * NS
K = 128
NCHUNK = 80
EP = NW * NCHUNK * K
ET = EP // NW
RPT = NP // NS
ZR = 128


@functools.lru_cache(maxsize=None)
def _make_sc_agg(D: int, n_pre: int, with_counts: bool):
  mesh = plsc.VectorSubcoreMesh(
      core_axis_name="c", subcore_axis_name="s", num_cores=NC,
      num_subcores=NS)

  out_type = [jax.ShapeDtypeStruct((NC, NP, D), jnp.float32)] * n_pre
  if with_counts:
    out_type.append(jax.ShapeDtypeStruct((NC, NP, 16), jnp.float32))

  scratch_types = [
      pltpu.VMEM_SHARED((NP, D), jnp.float32),
      pltpu.VMEM((NCHUNK, K), jnp.int32),
      pltpu.VMEM((NCHUNK, K), jnp.int32),
      pltpu.VMEM((4, K, D), jnp.float32),
      pltpu.VMEM((ZR, D), jnp.float32),
      pltpu.SemaphoreType.DMA,
      pltpu.SemaphoreType.DMA,
      pltpu.SemaphoreType.DMA,
      pltpu.SemaphoreType.DMA,
      pltpu.SemaphoreType.DMA,
      pltpu.SemaphoreType.DMA,
      pltpu.SemaphoreType.DMA,
      pltpu.SemaphoreType.DMA,
  ]
  if with_counts:
    scratch_types += [
        pltpu.VMEM((K, 16), jnp.float32),
        pltpu.VMEM((ZR, 16), jnp.float32),
        pltpu.VMEM_SHARED((NP, 16), jnp.float32),
    ]

  def body(*refs):
    pres = refs[:n_pre]
    src, dst, zeros = refs[n_pre:n_pre + 3]
    k = n_pre + 3
    if with_counts:
      zeros16, ones = refs[k:k + 2]
      k += 2
    outs = refs[k:k + n_pre]
    k += n_pre
    if with_counts:
      out_cnt = refs[k]
      k += 1
    acc, idx_s, idx_d, rows, zbuf = refs[k:k + 5]
    k += 5
    gsem = refs[k:k + 4]
    ssem = refs[k + 4:k + 8]
    k += 8
    if with_counts:
      ones_v, zbuf16, cnt = refs[k:k + 3]

    c = lax.axis_index("c")
    s = lax.axis_index("s")
    wid = s * NC + c
    rbase = s * RPT

    pltpu.sync_copy(src.at[wid], idx_s)
    pltpu.sync_copy(dst.at[wid], idx_d)
    if with_counts:
      pltpu.sync_copy(ones, ones_v)

    for phase in range(n_pre):
      pre = pres[phase]
      do_cnt = with_counts and phase == 0

      pltpu.sync_copy(zeros, zbuf)
      if do_cnt:
        pltpu.sync_copy(zeros16, zbuf16)
      for j in range(RPT // ZR):
        pltpu.sync_copy(zbuf, acc.at[pl.ds(rbase + j * ZR, ZR)])
        if do_cnt:
          pltpu.sync_copy(zbuf16, cnt.at[pl.ds(rbase + j * ZR, ZR)])
      plsc.subcore_barrier()

      def gather(g, j):
        pltpu.async_copy(pre.at[idx_s.at[g]], rows.at[j], gsem[j])

      def gwait(j):
        pltpu.make_async_copy(pre.at[idx_s.at[0]], rows.at[j],
                              gsem[j]).wait()

      def scat(g, j):
        pltpu.async_copy(rows.at[j], acc.at[idx_d.at[g]], ssem[j], add=True)
        if do_cnt:
          pltpu.async_copy(ones_v, cnt.at[idx_d.at[g]], ssem[j], add=True)

      def swait(j):
        pltpu.make_async_copy(rows.at[j], acc.at[idx_d.at[0]],
                              ssem[j]).wait()
        if do_cnt:
          pltpu.make_async_copy(ones_v, cnt.at[idx_d.at[0]],
                                ssem[j]).wait()

      gather(0, 0)
      gather(1, 1)
      gwait(0); scat(0, 0); gather(2, 2)
      gwait(1); scat(1, 1); gather(3, 3)
      gwait(2); scat(2, 2); swait(0); gather(4, 0)
      gwait(3); scat(3, 3); swait(1); gather(5, 1)

      def round4(r, carry):
        for j in range(4):
          g = 4 * r + j
          gwait(j)
          scat(g, j)
          swait((j + 2) % 4)
          gather(g + 2, (j + 2) % 4)
        return carry

      lax.fori_loop(1, NCHUNK // 4 - 1, round4, 0)
      ge = NCHUNK - 4
      gwait(0); scat(ge, 0); swait(2); gather(ge + 2, 2)
      gwait(1); scat(ge + 1, 1); swait(3); gather(ge + 3, 3)
      gwait(2); scat(ge + 2, 2); swait(0)
      gwait(3); scat(ge + 3, 3); swait(1)
      swait(2)
      swait(3)
      plsc.subcore_barrier()

      for j in range(RPT // ZR):
        pltpu.sync_copy(acc.at[pl.ds(rbase + j * ZR, ZR)], zbuf)
        pltpu.sync_copy(zbuf, outs[phase].at[c, pl.ds(rbase + j * ZR, ZR)])
        if do_cnt:
          pltpu.sync_copy(cnt.at[pl.ds(rbase + j * ZR, ZR)], zbuf16)
          pltpu.sync_copy(zbuf16,
                          out_cnt.at[c, pl.ds(rbase + j * ZR, ZR)])
      if phase + 1 < n_pre:
        plsc.subcore_barrier()

  return pl.kernel(body, out_type=tuple(out_type), mesh=mesh,
                   scratch_types=scratch_types,
                   compiler_params=pltpu.CompilerParams(
                       use_tc_tiling_on_sc=False))


_BM = 2560


def _tc1_body(x_ref, wl_ref, wr_ref, b_ref, prea_ref, preb_ref, r_ref):
  xb = x_ref[...]
  pre = jnp.dot(xb, wl_ref[...], preferred_element_type=jnp.float32)
  prea_ref[...] = pre[:, :64]
  preb_ref[...] = pre[:, 64:]
  r_ref[...] = (jnp.dot(xb, wr_ref[...], preferred_element_type=jnp.float32)
                + b_ref[...])


def _make_tc_mid_body(n_parts):
  def tc_mid_body(*refs):
    part_refs = refs[:n_parts]
    pcnt_ref, r_ref, wl_ref, wr_ref, b_ref, pre_ref, rn_ref = refs[n_parts:]
    cnt = jnp.maximum(pcnt_ref[0, :, 0:1] + pcnt_ref[1, :, 0:1], 1.0)
    mean = jnp.concatenate([p[0] + p[1] for p in part_refs], axis=1) / cnt
    h = jnp.maximum(mean + r_ref[...], 0.0)
    pre_ref[...] = jnp.dot(h, wl_ref[...], preferred_element_type=jnp.float32)
    rn_ref[...] = (jnp.dot(h, wr_ref[...],
                           preferred_element_type=jnp.float32) + b_ref[...])
  return tc_mid_body


def _tc4_body(part_ref, pcnt_ref, r_ref, out_ref):
  p = part_ref[0, :, 0:1] + part_ref[1, :, 0:1]
  cnt = jnp.maximum(pcnt_ref[0, :, 0:1] + pcnt_ref[1, :, 0:1], 1.0)
  out_ref[...] = p / cnt + r_ref[...]


def _row_spec(d):
  return pl.BlockSpec((_BM, d), lambda i: (i, 0))


def _part_spec(d):
  return pl.BlockSpec((NC, _BM, d), lambda i: (0, i, 0))


def _full_spec(a, b):
  return pl.BlockSpec((a, b), lambda i: (0, 0))


def _tc1(x, wl, wr, b):
  return pl.pallas_call(
      _tc1_body,
      grid=(NP // _BM,),
      in_specs=[_row_spec(128), _full_spec(128, 128),
                _full_spec(128, 128), _full_spec(1, 128)],
      out_specs=[_row_spec(64), _row_spec(64), _row_spec(128)],
      out_shape=[jax.ShapeDtypeStruct((N, 64), jnp.float32),
                 jax.ShapeDtypeStruct((N, 64), jnp.float32),
                 jax.ShapeDtypeStruct((N, 128), jnp.float32)],
  )(x, wl, wr, b)


def _tc_mid(parts, pcnt, r, wl, wr, b):
  d_in, d_out_l = wl.shape
  d_out_r = wr.shape[1]
  dp = d_in // len(parts)
  return pl.pallas_call(
      _make_tc_mid_body(len(parts)),
      grid=(NP // _BM,),
      in_specs=[_part_spec(dp)] * len(parts)
      + [_part_spec(16), _row_spec(d_in), _full_spec(d_in, d_out_l),
         _full_spec(d_in, d_out_r), _full_spec(1, d_out_r)],
      out_specs=[_row_spec(d_out_l), _row_spec(d_out_r)],
      out_shape=[jax.ShapeDtypeStruct((N, d_out_l), jnp.float32),
                 jax.ShapeDtypeStruct((N, d_out_r), jnp.float32)],
  )(*parts, pcnt, r, wl, wr, b)


def _tc4(part, pcnt, r):
  return pl.pallas_call(
      _tc4_body,
      grid=(NP // _BM,),
      in_specs=[_part_spec(16), _part_spec(16), _row_spec(1)],
      out_specs=_row_spec(1),
      out_shape=jax.ShapeDtypeStruct((N, 1), jnp.float32),
  )(part, pcnt, r)


def kernel(x, edge_index, Wl1, bl1, Wr1, Wl2, bl2, Wr2, Wl3, bl3, Wr3):
  pad = EP - E
  src = jnp.concatenate(
      [edge_index[0].astype(jnp.int32), jnp.zeros((pad,), jnp.int32)]
  ).reshape(NW, NCHUNK, K)
  dst = jnp.concatenate(
      [edge_index[1].astype(jnp.int32), jnp.full((pad,), N, jnp.int32)]
  ).reshape(NW, NCHUNK, K)

  z64 = jnp.zeros((ZR, 64), jnp.float32)
  z16 = jnp.zeros((ZR, 16), jnp.float32)
  ones = jnp.ones((K, 16), jnp.float32)

  pre1a, pre1b, r1 = _tc1(x, Wl1, Wr1, bl1.reshape(1, -1))
  part1a, part1b, pcnt = _make_sc_agg(64, 2, True)(
      pre1a, pre1b, src, dst, z64, z16, ones)

  pre2, r2 = _tc_mid([part1a, part1b], pcnt, r1, Wl2, Wr2, bl2.reshape(1, -1))
  (part2,) = _make_sc_agg(64, 1, False)(pre2, src, dst, z64)

  wl3p = jnp.concatenate([Wl3, jnp.zeros((Wl3.shape[0], 15), jnp.float32)],
                         axis=1)
  pr3, rr3 = _tc_mid([part2], pcnt, r2, wl3p, Wr3, bl3.reshape(1, 1))
  (part3,) = _make_sc_agg(16, 1, False)(pr3, src, dst, z16)

  return _tc4(part3, pcnt, rr3)


# speedup vs baseline: 1.9144x; 1.9144x over previous
import functools

import jax
import jax.numpy as jnp
from jax import lax
from jax.experimental import pallas as pl
from jax.experimental.pallas import tpu as pltpu
from jax.experimental.pallas import tpu_sc as plsc

N = 10000
E = 320000
NP = 10240
NC = 2
NS = 16
NW = NC * NS
K = 128
NCHUNK = 80
EP = NW * NCHUNK * K
ET = EP // NW
RPT = NP // NS
ZR = 128


@functools.lru_cache(maxsize=None)
def _make_sc_counts():
  mesh = plsc.VectorSubcoreMesh(
      core_axis_name="c", subcore_axis_name="s", num_cores=NC,
      num_subcores=NS)

  scratch_types = [
      pltpu.VMEM_SHARED((NP, 16), jnp.float32),
      pltpu.VMEM((NCHUNK, K), jnp.int32),
      pltpu.VMEM((K, 16), jnp.float32),
      pltpu.VMEM((ZR, 16), jnp.float32),
      pltpu.SemaphoreType.DMA,
      pltpu.SemaphoreType.DMA,
      pltpu.SemaphoreType.DMA,
      pltpu.SemaphoreType.DMA,
  ]

  def body(dst, zeros16, ones, out_cnt, cnt, idx_d, ones_v, zbuf, *sems):
    c = lax.axis_index("c")
    s = lax.axis_index("s")
    wid = s * NC + c
    rbase = s * RPT

    pltpu.sync_copy(dst.at[wid], idx_d)
    pltpu.sync_copy(ones, ones_v)
    pltpu.sync_copy(zeros16, zbuf)
    for j in range(RPT // ZR):
      pltpu.sync_copy(zbuf, cnt.at[pl.ds(rbase + j * ZR, ZR)])
    plsc.subcore_barrier()

    def scat(g, j):
      pltpu.async_copy(ones_v, cnt.at[idx_d.at[g]], sems[j], add=True)

    def swait(j):
      pltpu.make_async_copy(ones_v, cnt.at[idx_d.at[0]], sems[j]).wait()

    for j in range(4):
      scat(j, j)

    def round4(r, carry):
      for j in range(4):
        swait(j)
        scat(4 * r + j, j)
      return carry

    lax.fori_loop(1, NCHUNK // 4, round4, 0)
    for j in range(4):
      swait(j)
    plsc.subcore_barrier()

    for j in range(RPT // ZR):
      pltpu.sync_copy(cnt.at[pl.ds(rbase + j * ZR, ZR)], zbuf)
      pltpu.sync_copy(zbuf, out_cnt.at[c, pl.ds(rbase + j * ZR, ZR)])

  return pl.kernel(body,
                   out_type=(jax.ShapeDtypeStruct((NC, NP, 16), jnp.float32),),
                   mesh=mesh, scratch_types=scratch_types,
                   compiler_params=pltpu.CompilerParams(
                       use_tc_tiling_on_sc=False))


@functools.lru_cache(maxsize=None)
def _make_sc_agg(D: int, n_pre: int):
  mesh = plsc.VectorSubcoreMesh(
      core_axis_name="c", subcore_axis_name="s", num_cores=NC,
      num_subcores=NS)

  out_type = [jax.ShapeDtypeStruct((NC, NP, D), jnp.float32)] * n_pre

  scratch_types = [
      pltpu.VMEM_SHARED((NP, D), jnp.float32),
      pltpu.VMEM_SHARED((NP, D), jnp.float32),
      pltpu.VMEM((NCHUNK, K), jnp.int32),
      pltpu.VMEM((NCHUNK, K), jnp.int32),
      pltpu.VMEM((2, K, D), jnp.float32),
      pltpu.SemaphoreType.DMA,
      pltpu.SemaphoreType.DMA,
      pltpu.SemaphoreType.DMA,
      pltpu.SemaphoreType.DMA,
  ]

  def body(*refs):
    pres = refs[:n_pre]
    src, dst, zeros = refs[n_pre:n_pre + 3]
    outs = refs[n_pre + 3:n_pre + 3 + n_pre]
    acc, table, idx_s, idx_d, rows = refs[2 * n_pre + 3:2 * n_pre + 8]
    gsem = refs[2 * n_pre + 8:2 * n_pre + 10]
    ssem = refs[2 * n_pre + 10:2 * n_pre + 12]

    c = lax.axis_index("c")
    s = lax.axis_index("s")
    wid = s * NC + c
    rbase = s * RPT

    pltpu.sync_copy(src.at[wid], idx_s)
    pltpu.sync_copy(dst.at[wid], idx_d)

    for phase in range(n_pre):
      pre = pres[phase]

      for j in range(RPT // ZR):
        sl = pl.ds(rbase + j * ZR, ZR)
        pltpu.sync_copy(pre.at[sl], rows.at[0])
        pltpu.sync_copy(rows.at[0], table.at[sl])
      pltpu.sync_copy(zeros, rows.at[0])
      for j in range(RPT // ZR):
        pltpu.sync_copy(rows.at[0], acc.at[pl.ds(rbase + j * ZR, ZR)])
      plsc.subcore_barrier()

      def gather(g, j):
        pltpu.async_copy(table.at[idx_s.at[g]], rows.at[j], gsem[j])

      def gwait(j):
        pltpu.make_async_copy(table.at[idx_s.at[0]], rows.at[j],
                              gsem[j]).wait()

      def scat(g, j):
        pltpu.async_copy(rows.at[j], acc.at[idx_d.at[g]], ssem[j], add=True)

      def swait(j):
        pltpu.make_async_copy(rows.at[j], acc.at[idx_d.at[0]],
                              ssem[j]).wait()

      gather(0, 0)
      gather(1, 1)

      def round2(r, carry):
        g = 2 * r
        gwait(0); scat(g, 0)
        gwait(1); scat(g + 1, 1)
        swait(0); gather(g + 2, 0)
        swait(1); gather(g + 3, 1)
        return carry

      lax.fori_loop(0, NCHUNK // 2 - 1, round2, 0)
      ge = NCHUNK - 2
      gwait(0); scat(ge, 0)
      gwait(1); scat(ge + 1, 1)
      swait(0)
      swait(1)
      plsc.subcore_barrier()

      for j in range(RPT // ZR):
        sl = pl.ds(rbase + j * ZR, ZR)
        pltpu.sync_copy(acc.at[sl], rows.at[0])
        pltpu.sync_copy(rows.at[0], outs[phase].at[c, sl])
      if phase + 1 < n_pre:
        plsc.subcore_barrier()

  return pl.kernel(body, out_type=tuple(out_type), mesh=mesh,
                   scratch_types=scratch_types,
                   compiler_params=pltpu.CompilerParams(
                       use_tc_tiling_on_sc=False))


_BM = 2560


def _tc1_body(x_ref, wl_ref, wr_ref, b_ref, prea_ref, preb_ref, r_ref):
  xb = x_ref[...]
  pre = jnp.dot(xb, wl_ref[...], preferred_element_type=jnp.float32)
  prea_ref[...] = pre[:, :64]
  preb_ref[...] = pre[:, 64:]
  r_ref[...] = (jnp.dot(xb, wr_ref[...], preferred_element_type=jnp.float32)
                + b_ref[...])


def _make_tc_mid_body(n_parts):
  def tc_mid_body(*refs):
    part_refs = refs[:n_parts]
    pcnt_ref, r_ref, wl_ref, wr_ref, b_ref, pre_ref, rn_ref = refs[n_parts:]
    cnt = jnp.maximum(pcnt_ref[0, :, 0:1] + pcnt_ref[1, :, 0:1], 1.0)
    mean = jnp.concatenate([p[0] + p[1] for p in part_refs], axis=1) / cnt
    h = jnp.maximum(mean + r_ref[...], 0.0)
    pre_ref[...] = jnp.dot(h, wl_ref[...], preferred_element_type=jnp.float32)
    rn_ref[...] = (jnp.dot(h, wr_ref[...],
                           preferred_element_type=jnp.float32) + b_ref[...])
  return tc_mid_body


def _tc4_body(part_ref, pcnt_ref, r_ref, out_ref):
  p = part_ref[0, :, 0:1] + part_ref[1, :, 0:1]
  cnt = jnp.maximum(pcnt_ref[0, :, 0:1] + pcnt_ref[1, :, 0:1], 1.0)
  out_ref[...] = p / cnt + r_ref[...]


def _row_spec(d):
  return pl.BlockSpec((_BM, d), lambda i: (i, 0))


def _part_spec(d):
  return pl.BlockSpec((NC, _BM, d), lambda i: (0, i, 0))


def _full_spec(a, b):
  return pl.BlockSpec((a, b), lambda i: (0, 0))


def _tc1(x, wl, wr, b):
  return pl.pallas_call(
      _tc1_body,
      grid=(NP // _BM,),
      in_specs=[_row_spec(128), _full_spec(128, 128),
                _full_spec(128, 128), _full_spec(1, 128)],
      out_specs=[_row_spec(64), _row_spec(64), _row_spec(128)],
      out_shape=[jax.ShapeDtypeStruct((N, 64), jnp.float32),
                 jax.ShapeDtypeStruct((N, 64), jnp.float32),
                 jax.ShapeDtypeStruct((N, 128), jnp.float32)],
  )(x, wl, wr, b)


def _tc_mid(parts, pcnt, r, wl, wr, b):
  d_in, d_out_l = wl.shape
  d_out_r = wr.shape[1]
  dp = d_in // len(parts)
  return pl.pallas_call(
      _make_tc_mid_body(len(parts)),
      grid=(NP // _BM,),
      in_specs=[_part_spec(dp)] * len(parts)
      + [_part_spec(16), _row_spec(d_in), _full_spec(d_in, d_out_l),
         _full_spec(d_in, d_out_r), _full_spec(1, d_out_r)],
      out_specs=[_row_spec(d_out_l), _row_spec(d_out_r)],
      out_shape=[jax.ShapeDtypeStruct((N, d_out_l), jnp.float32),
                 jax.ShapeDtypeStruct((N, d_out_r), jnp.float32)],
  )(*parts, pcnt, r, wl, wr, b)


def _tc4(part, pcnt, r):
  return pl.pallas_call(
      _tc4_body,
      grid=(NP // _BM,),
      in_specs=[_part_spec(16), _part_spec(16), _row_spec(1)],
      out_specs=_row_spec(1),
      out_shape=jax.ShapeDtypeStruct((N, 1), jnp.float32),
  )(part, pcnt, r)


def _pad_rows(a):
  return jnp.concatenate(
      [a, jnp.zeros((NP - N, a.shape[1]), jnp.float32)], axis=0)


def kernel(x, edge_index, Wl1, bl1, Wr1, Wl2, bl2, Wr2, Wl3, bl3, Wr3):
  pad = EP - E
  src = jnp.concatenate(
      [edge_index[0].astype(jnp.int32), jnp.zeros((pad,), jnp.int32)]
  ).reshape(NW, NCHUNK, K)
  dst = jnp.concatenate(
      [edge_index[1].astype(jnp.int32), jnp.full((pad,), N, jnp.int32)]
  ).reshape(NW, NCHUNK, K)

  z64 = jnp.zeros((ZR, 64), jnp.float32)
  z16 = jnp.zeros((ZR, 16), jnp.float32)
  ones = jnp.ones((K, 16), jnp.float32)

  (pcnt,) = _make_sc_counts()(dst, z16, ones)

  pre1a, pre1b, r1 = _tc1(x, Wl1, Wr1, bl1.reshape(1, -1))
  part1a, part1b = _make_sc_agg(64, 2)(
      _pad_rows(pre1a), _pad_rows(pre1b), src, dst, z64)

  pre2, r2 = _tc_mid([part1a, part1b], pcnt, r1, Wl2, Wr2, bl2.reshape(1, -1))
  (part2,) = _make_sc_agg(64, 1)(_pad_rows(pre2), src, dst, z64)

  wl3p = jnp.concatenate([Wl3, jnp.zeros((Wl3.shape[0], 15), jnp.float32)],
                         axis=1)
  pr3, rr3 = _tc_mid([part2], pcnt, r2, wl3p, Wr3, bl3.reshape(1, 1))
  (part3,) = _make_sc_agg(16, 1)(_pad_rows(pr3), src, dst, z16)

  return _tc4(part3, pcnt, rr3)
